# baseline (speedup 1.0000x reference)
import jax
import jax.numpy as jnp
from jax.experimental import pallas as pl
from jax.experimental.pallas import tpu as pltpu

N = 10000
D = 128
C = 16
BNW = 1000
GRID = N // (2 * BNW)

_SELU_ALPHA = 1.6732632423543772848170429916717
_SELU_SCALE = 1.0507009873554804934193349852946


def _chain(x, w, b2):
    logits = jnp.dot(x, w, preferred_element_type=jnp.float32)
    lt = logits.T + b2
    m = jnp.max(lt, axis=0, keepdims=True)
    e = jnp.exp(lt - m)
    at = e / jnp.sum(e, axis=0, keepdims=True)
    return at, jnp.sum(at, axis=1, keepdims=True)


def _phase1(x0_ref, x1_ref, w_ref, b_ref, part_ref, assign_ref, sz_ref):
    w = w_ref[...]
    b2 = b_ref[...]
    x0 = x0_ref[...]
    x1 = x1_ref[...]

    at0, s0 = _chain(x0, w, b2)
    at1, s1 = _chain(x1, w, b2)

    assign_ref[0:BNW, :] = at0.T
    assign_ref[BNW:2 * BNW, :] = at1.T

    part_ref[0, :, :] = jax.lax.dot_general(
        at0, x0, (((1,), (0,)), ((), ())),
        preferred_element_type=jnp.float32,
    ) + jax.lax.dot_general(
        at1, x1, (((1,), (0,)), ((), ())),
        preferred_element_type=jnp.float32,
    )
    sz_ref[0, :, :] = s0 + s1


def _phase2(part_ref, sz_ref, out_ref):
    pooled = jnp.sum(part_ref[...], axis=0) / jnp.sum(sz_ref[...], axis=0)
    out_ref[...] = _SELU_SCALE * jnp.where(
        pooled > 0, pooled, _SELU_ALPHA * (jnp.exp(pooled) - 1.0)
    )


def kernel(features, edge_index, W, b):
    del edge_index
    b2 = b.reshape(C, 1)
    partials, assignments, sizes = pl.pallas_call(
        _phase1,
        grid=(GRID,),
        in_specs=[
            pl.BlockSpec((BNW, D), lambda i: (2 * i, 0)),
            pl.BlockSpec((BNW, D), lambda i: (2 * i + 1, 0)),
            pl.BlockSpec((D, C), lambda i: (0, 0)),
            pl.BlockSpec((C, 1), lambda i: (0, 0)),
        ],
        out_specs=[
            pl.BlockSpec((1, C, D), lambda i: (i, 0, 0)),
            pl.BlockSpec((2 * BNW, C), lambda i: (i, 0)),
            pl.BlockSpec((1, C, 1), lambda i: (i, 0, 0)),
        ],
        out_shape=[
            jax.ShapeDtypeStruct((GRID, C, D), jnp.float32),
            jax.ShapeDtypeStruct((N, C), jnp.float32),
            jax.ShapeDtypeStruct((GRID, C, 1), jnp.float32),
        ],
        compiler_params=pltpu.CompilerParams(
            dimension_semantics=("parallel",),
        ),
    )(features, features, W, b2)

    features_pooled = pl.pallas_call(
        _phase2,
        out_shape=jax.ShapeDtypeStruct((C, D), jnp.float32),
    )(partials, sizes)
    return (features_pooled, assignments)
